# Initial kernel scaffold; baseline (speedup 1.0000x reference)
import jax
import jax.numpy as jnp
from jax.experimental import pallas as pl


def kernel(predict_bboxes, predict_labels, gt_bboxes, gt_labels, neg_pos_ratio):
    raise NotImplementedError("write your pallas kernel here")


# speedup vs baseline: 2.0579x; 2.0579x over previous
import jax
import jax.numpy as jnp
from jax import lax
from jax.experimental import pallas as pl


_NEG_INF = float("-inf")


def _row_pass(lab_ref, gt_ref, pb_ref, gb_ref, ls_ref, stats_ref):
    P, C = lab_ref.shape[1], lab_ref.shape[2]
    x = lab_ref[0]
    e = jnp.exp(x)
    s = jnp.sum(e, axis=1)
    lse = jnp.log(s)
    gt = gt_ref[0, 0]
    cio = lax.broadcasted_iota(jnp.int32, (P, C), 1)
    g = jnp.sum(jnp.where(cio == gt[:, None], x, 0.0), axis=1)
    ce = lse - g
    pos = gt > 0
    posf = pos.astype(jnp.float32)
    pos_cnt = jnp.sum(posf)
    ce_pos = jnp.sum(jnp.where(pos, ce, 0.0))
    lsneg = jnp.where(pos, _NEG_INF, ce)

    pb = pb_ref[0]
    gb = gb_ref[0]
    d = jnp.abs(pb - gb)
    sl = jnp.where(d < 1.0, 0.5 * d * d, d - 0.5)
    loc = jnp.sum(sl * posf[None, :])

    ls_ref[0, 0] = lsneg
    lane = lax.broadcasted_iota(jnp.int32, (128,), 0)
    stats_ref[0, 0] = jnp.where(
        lane == 0, pos_cnt,
        jnp.where(lane == 1, ce_pos, jnp.where(lane == 2, loc, 0.0)))


def _select(ls_ref, st_ref, npr_ref, out_ref):
    Brows, P = ls_ref.shape
    LS = ls_ref[...]
    K = lax.bitcast_convert_type(LS, jnp.int32)
    K = K ^ (jnp.int32(0x7FFFFFFF) & (K >> 31))
    st = st_ref[...]
    pos_cnt = st[:, 0:1]
    ce_pos = st[:, 1:2]
    loc_sum = st[:, 2:3]
    npr = npr_ref[0, 0]
    pos_i = pos_cnt.astype(jnp.int32)
    neg_i = P - pos_i
    k = jnp.minimum(npr * pos_i, neg_i)

    lo = jnp.full((Brows, 1), -2**31, jnp.int32)
    hi = jnp.full((Brows, 1), 2**31 - 1, jnp.int32)

    def it(_, lohi):
        lo, hi = lohi
        xr = lo ^ hi
        mid = (lo & hi) + (xr >> 1) + (xr & 1)
        cnt = jnp.sum((K >= mid).astype(jnp.int32), axis=1, keepdims=True)
        ge = cnt >= k
        return jnp.where(ge, mid, lo), jnp.where(ge, hi, mid - 1)

    lo, hi = lax.fori_loop(0, 32, it, (lo, hi))
    T = lo
    tb = T ^ (jnp.int32(0x7FFFFFFF) & (T >> 31))
    thr = lax.bitcast_convert_type(tb, jnp.float32)
    gtm = K > T
    cnt_gt = jnp.sum(gtm.astype(jnp.int32), axis=1, keepdims=True)
    sum_gt = jnp.sum(jnp.where(gtm, LS, 0.0), axis=1, keepdims=True)
    topk = jnp.where(
        k > 0, sum_gt + (k - cnt_gt).astype(jnp.float32) * thr, 0.0)

    conf_total = jnp.sum(ce_pos + topk)
    loc_total = jnp.sum(loc_sum)
    denom = jnp.sum(pos_cnt) + 1e-7
    lane = lax.broadcasted_iota(jnp.int32, (128,), 0)
    out_ref[0] = jnp.where(
        lane == 0, loc_total / denom,
        jnp.where(lane == 1, conf_total / denom, 0.0))


def _ssd_loss(predict_bboxes, predict_labels, gt_bboxes, gt_labels, npr):
    B, P, C = predict_labels.shape
    pb = jnp.transpose(predict_bboxes, (0, 2, 1))
    gb = jnp.transpose(gt_bboxes, (0, 2, 1))
    gt3 = gt_labels.reshape(B, 1, P)
    ls, stats = pl.pallas_call(
        _row_pass,
        grid=(B,),
        in_specs=[
            pl.BlockSpec((1, P, C), lambda b: (b, 0, 0)),
            pl.BlockSpec((1, 1, P), lambda b: (b, 0, 0)),
            pl.BlockSpec((1, 4, P), lambda b: (b, 0, 0)),
            pl.BlockSpec((1, 4, P), lambda b: (b, 0, 0)),
        ],
        out_specs=[
            pl.BlockSpec((1, 1, P), lambda b: (b, 0, 0)),
            pl.BlockSpec((1, 1, 128), lambda b: (b, 0, 0)),
        ],
        out_shape=[
            jax.ShapeDtypeStruct((B, 1, P), jnp.float32),
            jax.ShapeDtypeStruct((B, 1, 128), jnp.float32),
        ],
    )(predict_labels, gt3, pb, gb)
    out = pl.pallas_call(
        _select,
        in_specs=[
            pl.BlockSpec((B, P), lambda: (0, 0)),
            pl.BlockSpec((B, 128), lambda: (0, 0)),
            pl.BlockSpec((1, 1), lambda: (0, 0)),
        ],
        out_specs=pl.BlockSpec((1, 128), lambda: (0, 0)),
        out_shape=jax.ShapeDtypeStruct((1, 128), jnp.float32),
    )(ls.reshape(B, P), stats.reshape(B, 128), npr.reshape(1, 1))
    return out[0, 0], out[0, 1]


def kernel(predict_bboxes, predict_labels, gt_bboxes, gt_labels,
           neg_pos_ratio):
    npr = jnp.asarray(neg_pos_ratio, jnp.int32)
    return _ssd_loss(predict_bboxes, predict_labels, gt_bboxes, gt_labels,
                     npr)
